# baseline (speedup 1.0000x reference)
import functools

import jax
import jax.numpy as jnp
from jax import lax
from jax.experimental import pallas as pl
from jax.experimental.pallas import tpu as pltpu
from jax.experimental.pallas import tpu_sc as plsc

PAD = 1
EPS = 1e-5


def _posids_body(S, ids_full_ref, ids_blk_ref, out_ref):
    j = pl.program_id(0)
    CH = out_ref.shape[1]
    m_full = (ids_full_ref[...] != PAD).astype(jnp.bfloat16)
    r = lax.broadcasted_iota(jnp.int32, (S, CH), 0)
    c = lax.broadcasted_iota(jnp.int32, (S, CH), 1) + j * CH
    upper = (r <= c).astype(jnp.bfloat16)
    incr = lax.dot_general(
        m_full, upper, (((1,), (0,)), ((), ())),
        preferred_element_type=jnp.float32)
    mb = (ids_blk_ref[...] != PAD).astype(jnp.int32)
    out_ref[...] = incr.astype(jnp.int32) * mb + PAD


def _position_ids(input_ids):
    B, S = input_ids.shape
    CH = 256
    grid = (S // CH,)
    return pl.pallas_call(
        functools.partial(_posids_body, S),
        grid=grid,
        in_specs=[
            pl.BlockSpec((B, S), lambda j: (0, 0)),
            pl.BlockSpec((B, CH), lambda j: (0, j)),
        ],
        out_specs=pl.BlockSpec((B, CH), lambda j: (0, j)),
        out_shape=jax.ShapeDtypeStruct((B, S), jnp.int32),
    )(input_ids, input_ids)


_NW = 32
_CHUNK = 32


def _sc_gather(word_emb, pos_emb, idw, idp):
    n = idw.shape[0]
    hid = word_emb.shape[1]
    per = n // _NW
    nchunks = per // _CHUNK
    mesh = plsc.VectorSubcoreMesh(core_axis_name="c", subcore_axis_name="s")

    @functools.partial(
        pl.kernel,
        out_type=jax.ShapeDtypeStruct((n, hid), jnp.float32),
        mesh=mesh,
        scratch_types=[
            pltpu.VMEM((per,), jnp.int32),
            pltpu.VMEM((per,), jnp.int32),
            pltpu.VMEM((_CHUNK, hid), jnp.float32),
            pltpu.VMEM((_CHUNK, hid), jnp.float32),
            pltpu.VMEM((_CHUNK, hid), jnp.float32),
            pltpu.VMEM((_CHUNK, hid), jnp.float32),
            pltpu.SemaphoreType.DMA,
            pltpu.SemaphoreType.DMA,
            pltpu.SemaphoreType.DMA,
        ],
    )
    def k(word_hbm, pos_hbm, idw_hbm, idp_hbm, g_hbm,
          idw_v, idp_v, bufw0, bufw1, bufp0, bufp1,
          semgw, semgp, semst):
        wid = lax.axis_index("s") * 2 + lax.axis_index("c")
        base = wid * per
        pltpu.sync_copy(idw_hbm.at[pl.ds(base, per)], idw_v)
        pltpu.sync_copy(idp_hbm.at[pl.ds(base, per)], idp_v)
        bufw = [bufw0, bufw1]
        bufp = [bufp0, bufp1]
        gath = [None, None]
        st = [None, None]

        def issue_gather(c):
            b = c & 1
            gw = pltpu.async_copy(
                word_hbm.at[idw_v.at[pl.ds(c * _CHUNK, _CHUNK)]],
                bufw[b], semgw)
            gp = pltpu.async_copy(
                pos_hbm.at[idp_v.at[pl.ds(c * _CHUNK, _CHUNK)]],
                bufp[b], semgp)
            gath[b] = (gw, gp)

        issue_gather(0)
        for c in range(nchunks):
            b = c & 1
            gw, gp = gath[b]
            gw.wait()
            gp.wait()
            if c + 1 < nchunks:
                if c >= 1:
                    st[1 - b].wait()
                issue_gather(c + 1)
            bw, bp = bufw[b], bufp[b]

            @pl.loop(0, _CHUNK)
            def _row(r):
                @pl.loop(0, hid, step=16)
                def _col(kk):
                    bw[r, pl.ds(kk, 16)] = (
                        bw[r, pl.ds(kk, 16)] + bp[r, pl.ds(kk, 16)])

            st[b] = pltpu.async_copy(
                bufw[b], g_hbm.at[pl.ds(base + c * _CHUNK, _CHUNK)], semst)
        for h in st:
            if h is not None:
                h.wait()

    return k(word_emb, pos_emb, idw, idp)


_T = 256


def _combine_body(g_in_ref, d_ref, s_ref, tm_ref,
                  dhi_ref, dlo_ref, shi_ref, slo_ref,
                  tt_ref, g_ref, b_ref, out_ref):
    T = out_ref.shape[0]
    nd = dhi_ref.shape[0]
    ns = shi_ref.shape[0]

    ids_d = d_ref[0, 0, :]
    ids_s = s_ref[0, 0, :]
    tm = tm_ref[0, 0, :]

    dscale = ((ids_d != -1).astype(jnp.float32) * tm).astype(jnp.bfloat16)
    sscale = ((ids_s != -1).astype(jnp.float32) * tm).astype(jnp.bfloat16)
    d_idx = jnp.clip(ids_d, 0, nd - 1)
    s_idx = jnp.clip(ids_s, 0, ns - 1)
    ohd = ((lax.broadcasted_iota(jnp.int32, (nd, T), 0) == d_idx[None, :]
            ).astype(jnp.bfloat16)) * dscale[None, :]
    ohs = ((lax.broadcasted_iota(jnp.int32, (ns, T), 0) == s_idx[None, :]
            ).astype(jnp.bfloat16)) * sscale[None, :]
    dn = (((0,), (0,)), ((), ()))
    demb = (lax.dot_general(ohd, dhi_ref[...], dn, preferred_element_type=jnp.float32)
            + lax.dot_general(ohd, dlo_ref[...], dn, preferred_element_type=jnp.float32))
    semb = (lax.dot_general(ohs, shi_ref[...], dn, preferred_element_type=jnp.float32)
            + lax.dot_general(ohs, slo_ref[...], dn, preferred_element_type=jnp.float32))

    x = g_in_ref[...] + tt_ref[...] + demb + semb
    mu = jnp.mean(x, axis=-1, keepdims=True)
    xc = x - mu
    var = jnp.mean(xc * xc, axis=-1, keepdims=True)
    inv = lax.rsqrt(var + EPS)
    out_ref[...] = xc * inv * g_ref[...] + b_ref[...]


def _combine_half(prev, gsum, d3, s3, tm3, dhi, dlo, shi, slo,
                  tt, gamma, beta, half, n_total):
    nh, hid = gsum.shape
    nblk = nh // _T
    off = half * nblk
    nd = dhi.shape[0]
    ns = shi.shape[0]
    in_specs = [
        pl.BlockSpec((_T, hid), lambda i: (i, 0)),
        pl.BlockSpec((1, 1, _T), lambda i: (i + off, 0, 0)),
        pl.BlockSpec((1, 1, _T), lambda i: (i + off, 0, 0)),
        pl.BlockSpec((1, 1, _T), lambda i: (i + off, 0, 0)),
        pl.BlockSpec((nd, hid), lambda i: (0, 0)),
        pl.BlockSpec((nd, hid), lambda i: (0, 0)),
        pl.BlockSpec((ns, hid), lambda i: (0, 0)),
        pl.BlockSpec((ns, hid), lambda i: (0, 0)),
        pl.BlockSpec((1, hid), lambda i: (0, 0)),
        pl.BlockSpec((1, hid), lambda i: (0, 0)),
        pl.BlockSpec((1, hid), lambda i: (0, 0)),
    ]
    args = [gsum, d3, s3, tm3, dhi, dlo, shi, slo, tt, gamma, beta]
    aliases = {}
    if prev is not None:
        in_specs = [pl.BlockSpec(memory_space=pl.ANY)] + in_specs
        args = [prev] + args
        aliases = {0: 0}
    if prev is None:
        def body2(*refs):
            _combine_body(*refs)
    else:
        def body2(prev_ref, *refs):
            _combine_body(*refs)
    return pl.pallas_call(
        body2,
        grid=(nblk,),
        in_specs=in_specs,
        out_specs=pl.BlockSpec((_T, hid), lambda i: (i + off, 0)),
        out_shape=jax.ShapeDtypeStruct((n_total, hid), jnp.float32),
        input_output_aliases=aliases,
    )(*args)


def kernel(input_ids, depths, sibling_indices, tree_attention_mask,
           word_embeddings, position_embeddings, token_type_embeddings,
           depth_embeddings, sibling_index_embeddings, ln_gamma, ln_beta):
    B, S = input_ids.shape
    n = B * S
    nh = n // 2
    hid = word_embeddings.shape[1]

    input_ids = input_ids.astype(jnp.int32)
    position_ids = _position_ids(input_ids)

    idw = input_ids.reshape(n)
    idp = position_ids.reshape(n)

    dhi = depth_embeddings.astype(jnp.bfloat16)
    dlo = (depth_embeddings - dhi.astype(jnp.float32)).astype(jnp.bfloat16)
    shi = sibling_index_embeddings.astype(jnp.bfloat16)
    slo = (sibling_index_embeddings - shi.astype(jnp.float32)
           ).astype(jnp.bfloat16)

    nblk = n // _T
    d3 = depths.reshape(nblk, 1, _T).astype(jnp.int32)
    s3 = sibling_indices.reshape(nblk, 1, _T).astype(jnp.int32)
    tm3 = tree_attention_mask.reshape(nblk, 1, _T).astype(jnp.float32)
    tt = token_type_embeddings.astype(jnp.float32)
    g2 = ln_gamma.reshape(1, hid)
    b2 = ln_beta.reshape(1, hid)

    out = None
    for half in range(2):
        sl = slice(half * nh, (half + 1) * nh)
        gsum = _sc_gather(
            word_embeddings, position_embeddings, idw[sl], idp[sl])
        out = _combine_half(out, gsum, d3, s3, tm3, dhi, dlo, shi, slo,
                            tt, g2, b2, half, n)
    return out.reshape(B, S, hid)


# speedup vs baseline: 1.3187x; 1.3187x over previous
import functools

import jax
import jax.numpy as jnp
from jax import lax
from jax.experimental import pallas as pl
from jax.experimental.pallas import tpu as pltpu
from jax.experimental.pallas import tpu_sc as plsc

PAD = 1
EPS = 1e-5


def _posids_body(S, ids_full_ref, ids_blk_ref, out_ref):
    j = pl.program_id(0)
    CH = out_ref.shape[1]
    m_full = (ids_full_ref[...] != PAD).astype(jnp.bfloat16)
    r = lax.broadcasted_iota(jnp.int32, (S, CH), 0)
    c = lax.broadcasted_iota(jnp.int32, (S, CH), 1) + j * CH
    upper = (r <= c).astype(jnp.bfloat16)
    incr = lax.dot_general(
        m_full, upper, (((1,), (0,)), ((), ())),
        preferred_element_type=jnp.float32)
    mb = (ids_blk_ref[...] != PAD).astype(jnp.int32)
    out_ref[...] = incr.astype(jnp.int32) * mb + PAD


def _position_ids(input_ids):
    B, S = input_ids.shape
    CH = 256
    grid = (S // CH,)
    return pl.pallas_call(
        functools.partial(_posids_body, S),
        grid=grid,
        in_specs=[
            pl.BlockSpec((B, S), lambda j: (0, 0)),
            pl.BlockSpec((B, CH), lambda j: (0, j)),
        ],
        out_specs=pl.BlockSpec((B, CH), lambda j: (0, j)),
        out_shape=jax.ShapeDtypeStruct((B, S), jnp.int32),
    )(input_ids, input_ids)


_NW = 32
_CHUNK = 32


def _sc_gather(word_emb, pos_emb, idw, idp):
    n = idw.shape[0]
    hid = word_emb.shape[1]
    per = n // _NW
    nchunks = per // _CHUNK
    mesh = plsc.VectorSubcoreMesh(core_axis_name="c", subcore_axis_name="s")

    @functools.partial(
        pl.kernel,
        out_type=jax.ShapeDtypeStruct((n, hid), jnp.float32),
        mesh=mesh,
        scratch_types=[
            pltpu.VMEM((per,), jnp.int32),
            pltpu.VMEM((per,), jnp.int32),
            pltpu.VMEM((_CHUNK, hid), jnp.float32),
            pltpu.VMEM((_CHUNK, hid), jnp.float32),
            pltpu.VMEM((_CHUNK, hid), jnp.float32),
            pltpu.VMEM((_CHUNK, hid), jnp.float32),
            pltpu.SemaphoreType.DMA,
            pltpu.SemaphoreType.DMA,
            pltpu.SemaphoreType.DMA,
        ],
    )
    def k(word_hbm, pos_hbm, idw_hbm, idp_hbm, g_hbm,
          idw_v, idp_v, bufw0, bufw1, bufp0, bufp1,
          semgw, semgp, semst):
        wid = lax.axis_index("s") * 2 + lax.axis_index("c")
        base = wid * per
        pltpu.sync_copy(idw_hbm.at[pl.ds(base, per)], idw_v)
        pltpu.sync_copy(idp_hbm.at[pl.ds(base, per)], idp_v)
        bufw = [bufw0, bufw1]
        bufp = [bufp0, bufp1]
        gath = [None, None]
        st = [None, None]

        def issue_gather(c):
            b = c & 1
            gw = pltpu.async_copy(
                word_hbm.at[idw_v.at[pl.ds(c * _CHUNK, _CHUNK)]],
                bufw[b], semgw)
            gp = pltpu.async_copy(
                pos_hbm.at[idp_v.at[pl.ds(c * _CHUNK, _CHUNK)]],
                bufp[b], semgp)
            gath[b] = (gw, gp)

        issue_gather(0)
        for c in range(nchunks):
            b = c & 1
            gw, gp = gath[b]
            gw.wait()
            gp.wait()
            if c + 1 < nchunks:
                if c >= 1:
                    st[1 - b].wait()
                issue_gather(c + 1)
            bw, bp = bufw[b], bufp[b]

            @pl.loop(0, _CHUNK)
            def _row(r):
                for kk in range(0, hid, 16):
                    plsc.addupdate(bw.at[r, pl.ds(kk, 16)],
                                   bp[r, pl.ds(kk, 16)])

            st[b] = pltpu.async_copy(
                bufw[b], g_hbm.at[pl.ds(base + c * _CHUNK, _CHUNK)], semst)
        for h in st:
            if h is not None:
                h.wait()

    return k(word_emb, pos_emb, idw, idp)


_T = 256


def _combine_body(g_in_ref, d_ref, s_ref, tm_ref,
                  dhi_ref, dlo_ref, shi_ref, slo_ref,
                  tt_ref, g_ref, b_ref, out_ref):
    T = out_ref.shape[0]
    nd = dhi_ref.shape[0]
    ns = shi_ref.shape[0]

    ids_d = d_ref[0, 0, :]
    ids_s = s_ref[0, 0, :]
    tm = tm_ref[0, 0, :]

    dscale = ((ids_d != -1).astype(jnp.float32) * tm).astype(jnp.bfloat16)
    sscale = ((ids_s != -1).astype(jnp.float32) * tm).astype(jnp.bfloat16)
    d_idx = jnp.clip(ids_d, 0, nd - 1)
    s_idx = jnp.clip(ids_s, 0, ns - 1)
    ohd = ((lax.broadcasted_iota(jnp.int32, (nd, T), 0) == d_idx[None, :]
            ).astype(jnp.bfloat16)) * dscale[None, :]
    ohs = ((lax.broadcasted_iota(jnp.int32, (ns, T), 0) == s_idx[None, :]
            ).astype(jnp.bfloat16)) * sscale[None, :]
    dn = (((0,), (0,)), ((), ()))
    demb = (lax.dot_general(ohd, dhi_ref[...], dn, preferred_element_type=jnp.float32)
            + lax.dot_general(ohd, dlo_ref[...], dn, preferred_element_type=jnp.float32))
    semb = (lax.dot_general(ohs, shi_ref[...], dn, preferred_element_type=jnp.float32)
            + lax.dot_general(ohs, slo_ref[...], dn, preferred_element_type=jnp.float32))

    x = g_in_ref[...] + tt_ref[...] + demb + semb
    mu = jnp.mean(x, axis=-1, keepdims=True)
    xc = x - mu
    var = jnp.mean(xc * xc, axis=-1, keepdims=True)
    inv = lax.rsqrt(var + EPS)
    out_ref[...] = xc * inv * g_ref[...] + b_ref[...]


def _combine_half(prev, gsum, d3, s3, tm3, dhi, dlo, shi, slo,
                  tt, gamma, beta, half, n_total):
    nh, hid = gsum.shape
    nblk = nh // _T
    off = half * nblk
    nd = dhi.shape[0]
    ns = shi.shape[0]
    in_specs = [
        pl.BlockSpec((_T, hid), lambda i: (i, 0)),
        pl.BlockSpec((1, 1, _T), lambda i: (i + off, 0, 0)),
        pl.BlockSpec((1, 1, _T), lambda i: (i + off, 0, 0)),
        pl.BlockSpec((1, 1, _T), lambda i: (i + off, 0, 0)),
        pl.BlockSpec((nd, hid), lambda i: (0, 0)),
        pl.BlockSpec((nd, hid), lambda i: (0, 0)),
        pl.BlockSpec((ns, hid), lambda i: (0, 0)),
        pl.BlockSpec((ns, hid), lambda i: (0, 0)),
        pl.BlockSpec((1, hid), lambda i: (0, 0)),
        pl.BlockSpec((1, hid), lambda i: (0, 0)),
        pl.BlockSpec((1, hid), lambda i: (0, 0)),
    ]
    args = [gsum, d3, s3, tm3, dhi, dlo, shi, slo, tt, gamma, beta]
    aliases = {}
    if prev is not None:
        in_specs = [pl.BlockSpec(memory_space=pl.ANY)] + in_specs
        args = [prev] + args
        aliases = {0: 0}
    if prev is None:
        def body2(*refs):
            _combine_body(*refs)
    else:
        def body2(prev_ref, *refs):
            _combine_body(*refs)
    return pl.pallas_call(
        body2,
        grid=(nblk,),
        in_specs=in_specs,
        out_specs=pl.BlockSpec((_T, hid), lambda i: (i + off, 0)),
        out_shape=jax.ShapeDtypeStruct((n_total, hid), jnp.float32),
        input_output_aliases=aliases,
    )(*args)


def kernel(input_ids, depths, sibling_indices, tree_attention_mask,
           word_embeddings, position_embeddings, token_type_embeddings,
           depth_embeddings, sibling_index_embeddings, ln_gamma, ln_beta):
    B, S = input_ids.shape
    n = B * S
    nh = n // 2
    hid = word_embeddings.shape[1]

    input_ids = input_ids.astype(jnp.int32)
    position_ids = _position_ids(input_ids)

    idw = input_ids.reshape(n)
    idp = position_ids.reshape(n)

    dhi = depth_embeddings.astype(jnp.bfloat16)
    dlo = (depth_embeddings - dhi.astype(jnp.float32)).astype(jnp.bfloat16)
    shi = sibling_index_embeddings.astype(jnp.bfloat16)
    slo = (sibling_index_embeddings - shi.astype(jnp.float32)
           ).astype(jnp.bfloat16)

    nblk = n // _T
    d3 = depths.reshape(nblk, 1, _T).astype(jnp.int32)
    s3 = sibling_indices.reshape(nblk, 1, _T).astype(jnp.int32)
    tm3 = tree_attention_mask.reshape(nblk, 1, _T).astype(jnp.float32)
    tt = token_type_embeddings.astype(jnp.float32)
    g2 = ln_gamma.reshape(1, hid)
    b2 = ln_beta.reshape(1, hid)

    out = None
    for half in range(2):
        sl = slice(half * nh, (half + 1) * nh)
        gsum = _sc_gather(
            word_embeddings, position_embeddings, idw[sl], idp[sl])
        out = _combine_half(out, gsum, d3, s3, tm3, dhi, dlo, shi, slo,
                            tt, g2, b2, half, n)
    return out.reshape(B, S, hid)
